# baseline (speedup 1.0000x reference)
import jax
import jax.numpy as jnp
from jax import lax
from jax.experimental import pallas as pl
from jax.experimental.pallas import tpu as pltpu
from jax.experimental.pallas import tpu_sc as plsc

_THR = 0.3
_B, _C, _H, _W = 8, 768, 48, 48
_ROWS = _B * _C
_D = _H * _W
_NW = 32
_RPW = _ROWS // _NW
_CH = 24
_NCH = _RPW // _CH
_L = 16
_NG = _RPW // _L
_PAIR = (_B // 2) * _C


def _body(ca_hbm, fm_hbm, out_hbm, ca_v, idx_v, bufs, gs0, gs1, ws0, ws1):
    cid = lax.axis_index("c")
    sid = lax.axis_index("s")
    wid = sid * 2 + cid
    r0 = wid * _RPW
    i = r0 // _C
    imod = lax.rem(i, _B // 2)
    off = jnp.where(i < (_B // 2), _PAIR, -_PAIR)

    ca_off = imod * _C + (r0 - i * _C)
    pltpu.sync_copy(ca_hbm.at[pl.ds(ca_off, _RPW)], ca_v)

    for g in range(_NG):
        ca = ca_v[pl.ds(g * _L, _L)]
        base = r0 + g * _L + lax.iota(jnp.int32, _L)
        idx_v[pl.ds(g * _L, _L)] = base + jnp.where(ca < _THR, off, 0)

    gsems = [gs0, gs1]
    wsems = [ws0, ws1]
    gd, wd = {}, {}

    def start_gather(k):
        b = k % 2
        gd[k] = pltpu.async_copy(
            fm_hbm.at[idx_v.at[pl.ds(k * _CH, _CH)]], bufs.at[b], gsems[b])

    start_gather(0)
    for k in range(_NCH):
        b = k % 2
        if k + 1 < _NCH:
            if k - 1 >= 0:
                wd[k - 1].wait()
            start_gather(k + 1)
        gd[k].wait()
        wd[k] = pltpu.async_copy(
            bufs.at[b], out_hbm.at[pl.ds(r0 + k * _CH, _CH)], wsems[b])
    wd[_NCH - 2].wait()
    wd[_NCH - 1].wait()


def kernel(CA, feature_map):
    fm = feature_map.reshape(_ROWS, _D)
    ca = CA.reshape(-1)
    out = pl.kernel(
        _body,
        out_type=jax.ShapeDtypeStruct((_ROWS, _D), jnp.float32),
        mesh=plsc.VectorSubcoreMesh(core_axis_name="c", subcore_axis_name="s"),
        scratch_types=[
            pltpu.VMEM((_RPW,), jnp.float32),
            pltpu.VMEM((_RPW,), jnp.int32),
            pltpu.VMEM((2, _CH, _D), jnp.float32),
            pltpu.SemaphoreType.DMA,
            pltpu.SemaphoreType.DMA,
            pltpu.SemaphoreType.DMA,
            pltpu.SemaphoreType.DMA,
        ],
    )(ca, fm)
    return out.reshape(_B, _C, _H, _W)


# speedup vs baseline: 1.0024x; 1.0024x over previous
import jax
import jax.numpy as jnp
from jax import lax
from jax.experimental import pallas as pl
from jax.experimental.pallas import tpu as pltpu
from jax.experimental.pallas import tpu_sc as plsc

_THR = 0.3
_B, _C, _H, _W = 8, 768, 48, 48
_ROWS = _B * _C
_D = _H * _W
_NW = 32
_RPW = _ROWS // _NW
_CH = 24
_NCH = _RPW // _CH
_L = 16
_NG = _RPW // _L
_PAIR = (_B // 2) * _C


def _body(ca_hbm, fm_hbm, out_hbm, ca_v, idx_v, bufs, gs0, gs1, ws0, ws1):
    cid = lax.axis_index("c")
    sid = lax.axis_index("s")
    wid = sid * 2 + cid
    r0 = wid * _RPW
    i = r0 // _C
    imod = lax.rem(i, _B // 2)
    off = jnp.where(i < (_B // 2), _PAIR, -_PAIR)

    ca_off = imod * _C + (r0 - i * _C)
    pltpu.sync_copy(ca_hbm.at[pl.ds(ca_off, _RPW)], ca_v)

    for g in range(_NG):
        ca = ca_v[pl.ds(g * _L, _L)]
        base = r0 + g * _L + lax.iota(jnp.int32, _L)
        idx_v[pl.ds(g * _L, _L)] = base + jnp.where(ca < _THR, off, 0)

    gsems = [gs0, gs1]
    wsems = [ws0, ws1]
    gd, wd = {}, {}

    def start_gather(k):
        b = k % 2
        gd[k] = pltpu.async_copy(
            fm_hbm.at[idx_v.at[pl.ds(k * _CH, _CH)]], bufs.at[b], gsems[b])

    start_gather(0)
    for k in range(_NCH):
        b = k % 2
        if k + 1 < _NCH:
            if k - 1 >= 0:
                wd[k - 1].wait()
            start_gather(k + 1)
        gd[k].wait()
        wd[k] = pltpu.async_copy(
            bufs.at[b], out_hbm.at[pl.ds(r0 + k * _CH, _CH)], wsems[b])
    wd[_NCH - 2].wait()
    wd[_NCH - 1].wait()


def kernel(CA, feature_map):
    fm = feature_map.reshape(_ROWS, _D)
    ca = CA.reshape(-1)
    out = pl.kernel(
        _body,
        out_type=jax.ShapeDtypeStruct((_ROWS, _D), jnp.float32),
        mesh=plsc.VectorSubcoreMesh(core_axis_name="c", subcore_axis_name="s"),
        compiler_params=pltpu.CompilerParams(use_tc_tiling_on_sc=True),
        scratch_types=[
            pltpu.VMEM((_RPW,), jnp.float32),
            pltpu.VMEM((_RPW,), jnp.int32),
            pltpu.VMEM((2, _CH, _D), jnp.float32),
            pltpu.SemaphoreType.DMA,
            pltpu.SemaphoreType.DMA,
            pltpu.SemaphoreType.DMA,
            pltpu.SemaphoreType.DMA,
        ],
    )(ca, fm)
    return out.reshape(_B, _C, _H, _W)


# speedup vs baseline: 5.4736x; 5.4607x over previous
import jax
import jax.numpy as jnp
from jax import lax
from jax.experimental import pallas as pl
from jax.experimental.pallas import tpu as pltpu
from jax.experimental.pallas import tpu_sc as plsc

_THR = 0.3
_B, _C, _H, _W = 8, 768, 48, 48
_ROWS = _B * _H * _W
_HALF = _ROWS // 2
_NW = 32
_RPW = _HALF // _NW
_CH = 16
_NCH = _RPW // _CH
_L = 16
_NCG = _C // _L


def _body(ca_hbm, fm_hbm, out_hbm, ca_v,
          ir0, ir1, rgb0, rgb1, x10, x11, x20, x21,
          gi0, gi1, gr0, gr1, wi0, wi1, wr0, wr1):
    cid = lax.axis_index("c")
    sid = lax.axis_index("s")
    wid = sid * 2 + cid
    r0 = wid * _RPW
    b = r0 // (_H * _W)

    pltpu.sync_copy(ca_hbm.at[pl.ds(b * _C, _C)], ca_v)

    ir_bufs = [ir0, ir1]
    rgb_bufs = [rgb0, rgb1]
    x1_bufs = [x10, x11]
    x2_bufs = [x20, x21]
    gis = [gi0, gi1]
    grs = [gr0, gr1]
    wis = [wi0, wi1]
    wrs = [wr0, wr1]

    def start_loads(j, u):
        base = r0 + j * _CH
        pltpu.async_copy(fm_hbm.at[pl.ds(base, _CH)], ir_bufs[u], gis[u])
        pltpu.async_copy(fm_hbm.at[pl.ds(base + _HALF, _CH)], rgb_bufs[u],
                         grs[u])

    def wait_loads(u):
        pltpu.make_async_copy(fm_hbm.at[pl.ds(0, _CH)], ir_bufs[u],
                              gis[u]).wait()
        pltpu.make_async_copy(fm_hbm.at[pl.ds(0, _CH)], rgb_bufs[u],
                              grs[u]).wait()

    def start_stores(j, u):
        base = r0 + j * _CH
        pltpu.async_copy(x1_bufs[u], out_hbm.at[pl.ds(base, _CH)], wis[u])
        pltpu.async_copy(x2_bufs[u], out_hbm.at[pl.ds(base + _HALF, _CH)],
                         wrs[u])

    def wait_stores(u):
        pltpu.make_async_copy(x1_bufs[u], out_hbm.at[pl.ds(0, _CH)],
                              wis[u]).wait()
        pltpu.make_async_copy(x2_bufs[u], out_hbm.at[pl.ds(0, _CH)],
                              wrs[u]).wait()

    def compute(u):
        ir_b, rgb_b, x1_b, x2_b = ir_bufs[u], rgb_bufs[u], x1_bufs[u], x2_bufs[u]

        def cg_step(cg, _):
            c0 = cg * _L
            m = ca_v[pl.ds(c0, _L)] < _THR
            for r in range(_CH):
                a = ir_b[r, pl.ds(c0, _L)]
                g = rgb_b[r, pl.ds(c0, _L)]
                x1_b[r, pl.ds(c0, _L)] = jnp.where(m, g, a)
                x2_b[r, pl.ds(c0, _L)] = jnp.where(m, a, g)
            return _

        lax.fori_loop(0, _NCG, cg_step, 0)

    start_loads(0, 0)
    start_loads(1, 1)

    for j in (0, 1):
        u = j % 2
        wait_loads(u)
        compute(u)
        start_stores(j, u)
        start_loads(j + 2, u)

    def pair_step(i, _):
        k = 2 + 2 * i
        for bidx in (0, 1):
            j = k + bidx
            u = bidx
            wait_loads(u)
            wait_stores(u)
            compute(u)
            start_stores(j, u)
            start_loads(j + 2, u)
        return _

    lax.fori_loop(0, (_NCH - 4) // 2, pair_step, 0)

    for j in (_NCH - 2, _NCH - 1):
        u = j % 2
        wait_loads(u)
        wait_stores(u)
        compute(u)
        start_stores(j, u)
    wait_stores(0)
    wait_stores(1)


def kernel(CA, feature_map):
    fm = jnp.transpose(feature_map, (0, 2, 3, 1)).reshape(_ROWS, _C)
    ca = CA.reshape(-1)
    buf = lambda: pltpu.VMEM((_CH, _C), jnp.float32)
    out = pl.kernel(
        _body,
        out_type=jax.ShapeDtypeStruct((_ROWS, _C), jnp.float32),
        mesh=plsc.VectorSubcoreMesh(core_axis_name="c", subcore_axis_name="s"),
        compiler_params=pltpu.CompilerParams(use_tc_tiling_on_sc=True),
        scratch_types=[
            pltpu.VMEM((_C,), jnp.float32),
            buf(), buf(), buf(), buf(), buf(), buf(), buf(), buf(),
            pltpu.SemaphoreType.DMA, pltpu.SemaphoreType.DMA,
            pltpu.SemaphoreType.DMA, pltpu.SemaphoreType.DMA,
            pltpu.SemaphoreType.DMA, pltpu.SemaphoreType.DMA,
            pltpu.SemaphoreType.DMA, pltpu.SemaphoreType.DMA,
        ],
    )(ca, fm)
    return jnp.transpose(out.reshape(_B, _H, _W, _C), (0, 3, 1, 2))
